# baseline (speedup 1.0000x reference)
import jax
import jax.numpy as jnp
from jax import lax
from jax.experimental import pallas as pl
from jax.experimental.pallas import tpu as pltpu
from jax.experimental.pallas import tpu_sc as plsc
import functools

NC = 2
NS = 16
L = 16
NW = NC * NS

BATCH = 16384
EMB = 64
BPW = BATCH // NW
PPW = 2 * BPW
CHUNK = 128
UCH = BPW // CHUNK
VCH = PPW // CHUNK

_EPS2 = 1e-24


def _rsqrt_nr(x):
    i = lax.bitcast_convert_type(x, jnp.int32)
    i = jnp.int32(0x5F3759DF) - (i >> 1)
    y = lax.bitcast_convert_type(i, jnp.float32)
    for _ in range(3):
        y = y * (1.5 - 0.5 * x * y * y)
    return y


def _mf_body(uid_hbm, vid_hbm, u_hbm, v_hbm, out_hbm,
             idxu, idxv, urows, vrows, uu_a, uv0_a, uv1_a, vv0_a, vv1_a,
             acc_v, sem):
    wid = lax.axis_index("s") * NC + lax.axis_index("c")

    pltpu.sync_copy(uid_hbm.at[pl.ds(wid * UCH, UCH)], idxu)
    pltpu.sync_copy(vid_hbm.at[pl.ds(wid * VCH, VCH)], idxv)

    copies = []
    for j in range(UCH):
        copies.append(pltpu.async_copy(
            u_hbm.at[idxu.at[j]], urows.at[pl.ds(j * CHUNK, CHUNK)], sem))
    for j in range(VCH):
        copies.append(pltpu.async_copy(
            v_hbm.at[idxv.at[j]], vrows.at[pl.ds(j * CHUNK, CHUNK)], sem))
    for c in copies:
        c.wait()

    lane = lax.iota(jnp.int32, L)
    perm = [lane ^ s for s in (8, 4, 2, 1)]

    def allred(x):
        for idx in perm:
            x = x + x[idx]
        return x

    def dots(g, _):
        zero = jnp.zeros((L,), jnp.float32)
        uu_v = uv0_v = uv1_v = vv0_v = vv1_v = zero
        for j in range(L):
            i = g * L + j
            u = [urows[i, pl.ds(c * L, L)] for c in range(EMB // L)]
            v0 = [vrows[2 * i, pl.ds(c * L, L)] for c in range(EMB // L)]
            v1 = [vrows[2 * i + 1, pl.ds(c * L, L)] for c in range(EMB // L)]
            suu = u[0] * u[0]
            suv0 = u[0] * v0[0]
            suv1 = u[0] * v1[0]
            svv0 = v0[0] * v0[0]
            svv1 = v1[0] * v1[0]
            for c in range(1, EMB // L):
                suu += u[c] * u[c]
                suv0 += u[c] * v0[c]
                suv1 += u[c] * v1[c]
                svv0 += v0[c] * v0[c]
                svv1 += v1[c] * v1[c]
            m = lane == j
            uu_v = jnp.where(m, allred(suu), uu_v)
            uv0_v = jnp.where(m, allred(suv0), uv0_v)
            uv1_v = jnp.where(m, allred(suv1), uv1_v)
            vv0_v = jnp.where(m, allred(svv0), vv0_v)
            vv1_v = jnp.where(m, allred(svv1), vv1_v)
        uu_a[pl.ds(g * L, L)] = uu_v
        uv0_a[pl.ds(g * L, L)] = uv0_v
        uv1_a[pl.ds(g * L, L)] = uv1_v
        vv0_a[pl.ds(g * L, L)] = vv0_v
        vv1_a[pl.ds(g * L, L)] = vv1_v
        return 0

    lax.fori_loop(0, BPW // L, dots, 0, unroll=False)

    def norm_loss(k, acc):
        uu = jnp.maximum(uu_a[pl.ds(k * L, L)], _EPS2)
        vv0 = jnp.maximum(vv0_a[pl.ds(k * L, L)], _EPS2)
        vv1 = jnp.maximum(vv1_a[pl.ds(k * L, L)], _EPS2)
        ru = _rsqrt_nr(uu)
        p0 = uv0_a[pl.ds(k * L, L)] * ru * _rsqrt_nr(vv0)
        p1 = uv1_a[pl.ds(k * L, L)] * ru * _rsqrt_nr(vv1)
        e0 = p0 - 1.0
        return acc + e0 * e0 + p1 * p1

    acc = lax.fori_loop(0, BPW // L, norm_loss,
                        jnp.zeros((L,), jnp.float32), unroll=False)
    acc_v[...] = acc
    pltpu.sync_copy(acc_v, out_hbm.at[wid])


@jax.jit
def kernel(uid, items, U, V):
    uid2d = uid.reshape(NW * UCH, CHUNK)
    vid2d = items.reshape(NW * VCH, CHUNK)

    mf = pl.kernel(
        _mf_body,
        out_type=jax.ShapeDtypeStruct((NW, L), jnp.float32),
        mesh=plsc.VectorSubcoreMesh(core_axis_name="c", subcore_axis_name="s"),
        scratch_types=[
            pltpu.VMEM((UCH, CHUNK), jnp.int32),
            pltpu.VMEM((VCH, CHUNK), jnp.int32),
            pltpu.VMEM((BPW, EMB), jnp.float32),
            pltpu.VMEM((PPW, EMB), jnp.float32),
            pltpu.VMEM((BPW,), jnp.float32),
            pltpu.VMEM((BPW,), jnp.float32),
            pltpu.VMEM((BPW,), jnp.float32),
            pltpu.VMEM((BPW,), jnp.float32),
            pltpu.VMEM((BPW,), jnp.float32),
            pltpu.VMEM((L,), jnp.float32),
            pltpu.SemaphoreType.DMA,
        ],
        compiler_params=pltpu.CompilerParams(use_tc_tiling_on_sc=False),
    )
    partials = mf(uid2d, vid2d, U, V)
    return jnp.sum(partials) / jnp.float32(2 * BATCH)


# speedup vs baseline: 1.3399x; 1.3399x over previous
import jax
import jax.numpy as jnp
from jax import lax
from jax.experimental import pallas as pl
from jax.experimental.pallas import tpu as pltpu
from jax.experimental.pallas import tpu_sc as plsc

NC = 2
NS = 16
L = 16
NW = NC * NS

BATCH = 16384
EMB = 64
BPW = BATCH // NW
BLK = 16
NBLK = BPW // BLK

_EPS2 = 1e-24


def _rsqrt_nr(x):
    i = lax.bitcast_convert_type(x, jnp.int32)
    i = jnp.int32(0x5F3759DF) - (i >> 1)
    y = lax.bitcast_convert_type(i, jnp.float32)
    for _ in range(3):
        y = y * (1.5 - 0.5 * x * y * y)
    return y


def _mf_body(uid_hbm, vid_hbm, u_hbm, v_hbm, out_hbm,
             idxu, idxv, ubuf, vbuf, uu_a, uv0_a, uv1_a, vv0_a, vv1_a,
             acc_v, sem):
    wid = lax.axis_index("s") * NC + lax.axis_index("c")

    pltpu.sync_copy(uid_hbm.at[pl.ds(wid * BPW, BPW)], idxu)
    pltpu.sync_copy(vid_hbm.at[pl.ds(wid * 2 * BPW, 2 * BPW)], idxv)

    lane = lax.iota(jnp.int32, L)
    perm = [lane ^ s for s in (8, 4, 2, 1)]

    def allred(x):
        for idx in perm:
            x = x + x[idx]
        return x

    def block(t, _):
        uvec = idxu[pl.ds(t * BLK, L)]
        vvecA = idxv[pl.ds(t * 2 * BLK, L)]
        vvecB = idxv[pl.ds(t * 2 * BLK + L, L)]

        copies = []
        for j in range(BLK):
            r = uvec[j]
            roff = pl.multiple_of((r >> 3) * 8, 8)
            copies.append(pltpu.async_copy(
                u_hbm.at[pl.ds(roff, 8), :], ubuf.at[j], sem))
        for j in range(BLK):
            vv = vvecA if j < 8 else vvecB
            jj = j if j < 8 else j - 8
            for p in range(2):
                r = vv[2 * jj + p]
                roff = pl.multiple_of((r >> 3) * 8, 8)
                copies.append(pltpu.async_copy(
                    v_hbm.at[pl.ds(roff, 8), :], vbuf.at[2 * j + p], sem))
        for h in copies:
            h.wait()

        zero = jnp.zeros((L,), jnp.float32)
        uu_v = uv0_v = uv1_v = vv0_v = vv1_v = zero
        for j in range(BLK):
            vv = vvecA if j < 8 else vvecB
            jj = j if j < 8 else j - 8
            qu = uvec[j] & 7
            q0 = vv[2 * jj] & 7
            q1 = vv[2 * jj + 1] & 7
            u = [ubuf[j, qu, pl.ds(c * L, L)] for c in range(EMB // L)]
            v0 = [vbuf[2 * j, q0, pl.ds(c * L, L)] for c in range(EMB // L)]
            v1 = [vbuf[2 * j + 1, q1, pl.ds(c * L, L)] for c in range(EMB // L)]
            suu = u[0] * u[0]
            suv0 = u[0] * v0[0]
            suv1 = u[0] * v1[0]
            svv0 = v0[0] * v0[0]
            svv1 = v1[0] * v1[0]
            for c in range(1, EMB // L):
                suu += u[c] * u[c]
                suv0 += u[c] * v0[c]
                suv1 += u[c] * v1[c]
                svv0 += v0[c] * v0[c]
                svv1 += v1[c] * v1[c]
            m = lane == j
            uu_v = jnp.where(m, allred(suu), uu_v)
            uv0_v = jnp.where(m, allred(suv0), uv0_v)
            uv1_v = jnp.where(m, allred(suv1), uv1_v)
            vv0_v = jnp.where(m, allred(svv0), vv0_v)
            vv1_v = jnp.where(m, allred(svv1), vv1_v)
        uu_a[pl.ds(t * BLK, L)] = uu_v
        uv0_a[pl.ds(t * BLK, L)] = uv0_v
        uv1_a[pl.ds(t * BLK, L)] = uv1_v
        vv0_a[pl.ds(t * BLK, L)] = vv0_v
        vv1_a[pl.ds(t * BLK, L)] = vv1_v
        return 0

    lax.fori_loop(0, NBLK, block, 0, unroll=False)

    def norm_loss(k, acc):
        uu = jnp.maximum(uu_a[pl.ds(k * L, L)], _EPS2)
        vv0 = jnp.maximum(vv0_a[pl.ds(k * L, L)], _EPS2)
        vv1 = jnp.maximum(vv1_a[pl.ds(k * L, L)], _EPS2)
        ru = _rsqrt_nr(uu)
        p0 = uv0_a[pl.ds(k * L, L)] * ru * _rsqrt_nr(vv0)
        p1 = uv1_a[pl.ds(k * L, L)] * ru * _rsqrt_nr(vv1)
        e0 = p0 - 1.0
        return acc + e0 * e0 + p1 * p1

    acc = lax.fori_loop(0, BPW // L, norm_loss,
                        jnp.zeros((L,), jnp.float32), unroll=False)
    acc_v[...] = acc
    pltpu.sync_copy(acc_v, out_hbm.at[wid])


@jax.jit
def kernel(uid, items, U, V):
    vid = items.reshape(-1)

    mf = pl.kernel(
        _mf_body,
        out_type=jax.ShapeDtypeStruct((NW, L), jnp.float32),
        mesh=plsc.VectorSubcoreMesh(core_axis_name="c", subcore_axis_name="s"),
        scratch_types=[
            pltpu.VMEM((BPW,), jnp.int32),
            pltpu.VMEM((2 * BPW,), jnp.int32),
            pltpu.VMEM((BLK, 8, EMB), jnp.float32),
            pltpu.VMEM((2 * BLK, 8, EMB), jnp.float32),
            pltpu.VMEM((BPW,), jnp.float32),
            pltpu.VMEM((BPW,), jnp.float32),
            pltpu.VMEM((BPW,), jnp.float32),
            pltpu.VMEM((BPW,), jnp.float32),
            pltpu.VMEM((BPW,), jnp.float32),
            pltpu.VMEM((L,), jnp.float32),
            pltpu.SemaphoreType.DMA,
        ],
        compiler_params=pltpu.CompilerParams(use_tc_tiling_on_sc=True),
    )
    partials = mf(uid, vid, U, V)
    return jnp.sum(partials) / jnp.float32(2 * BATCH)


# speedup vs baseline: 1.3850x; 1.0337x over previous
import jax
import jax.numpy as jnp
from jax import lax
from jax.experimental import pallas as pl
from jax.experimental.pallas import tpu as pltpu
from jax.experimental.pallas import tpu_sc as plsc

NC = 2
NS = 16
L = 16
NW = NC * NS

BATCH = 16384
EMB = 64
BPW = BATCH // NW
BLK = 16
NBLK = BPW // BLK

_EPS2 = 1e-24


def _rsqrt_nr(x):
    i = lax.bitcast_convert_type(x, jnp.int32)
    i = jnp.int32(0x5F3759DF) - (i >> 1)
    y = lax.bitcast_convert_type(i, jnp.float32)
    for _ in range(3):
        y = y * (1.5 - 0.5 * x * y * y)
    return y


def _mf_body(uid_hbm, vid_hbm, u_hbm, v_hbm, out_hbm,
             idxu, idxv, ubuf, vbuf, uu_a, uv0_a, uv1_a, vv0_a, vv1_a,
             acc_v, sem0, sem1):
    wid = lax.axis_index("s") * NC + lax.axis_index("c")

    pltpu.sync_copy(uid_hbm.at[pl.ds(wid * BPW, BPW)], idxu)
    pltpu.sync_copy(vid_hbm.at[pl.ds(wid * 2 * BPW, 2 * BPW)], idxv)

    lane = lax.iota(jnp.int32, L)
    perm = [lane ^ s for s in (8, 4, 2, 1)]

    def allred(x):
        for idx in perm:
            x = x + x[idx]
        return x

    def issue(t, b, sem):
        uvec = idxu[pl.ds(t * BLK, L)]
        vvecA = idxv[pl.ds(t * 2 * BLK, L)]
        vvecB = idxv[pl.ds(t * 2 * BLK + L, L)]
        for j in range(BLK):
            r = uvec[j]
            roff = pl.multiple_of((r >> 3) * 8, 8)
            pltpu.async_copy(u_hbm.at[pl.ds(roff, 8), :], ubuf.at[b, j], sem)
        for j in range(BLK):
            vv = vvecA if j < 8 else vvecB
            jj = j if j < 8 else j - 8
            for p in range(2):
                r = vv[2 * jj + p]
                roff = pl.multiple_of((r >> 3) * 8, 8)
                pltpu.async_copy(v_hbm.at[pl.ds(roff, 8), :],
                                 vbuf.at[b, 2 * j + p], sem)

    def drain(b, sem):
        for j in range(BLK):
            pltpu.make_async_copy(u_hbm.at[pl.ds(0, 8), :],
                                  ubuf.at[b, j], sem).wait()
        for j in range(2 * BLK):
            pltpu.make_async_copy(v_hbm.at[pl.ds(0, 8), :],
                                  vbuf.at[b, j], sem).wait()

    issue(0, 0, sem0)

    def block(t, _):
        @pl.when(t + 1 < NBLK)
        def _():
            @pl.when((t & 1) == 0)
            def _():
                issue(t + 1, 1, sem1)

            @pl.when((t & 1) == 1)
            def _():
                issue(t + 1, 0, sem0)

        @pl.when((t & 1) == 0)
        def _():
            drain(0, sem0)

        @pl.when((t & 1) == 1)
        def _():
            drain(1, sem1)

        b = t & 1
        uvec = idxu[pl.ds(t * BLK, L)]
        vvecA = idxv[pl.ds(t * 2 * BLK, L)]
        vvecB = idxv[pl.ds(t * 2 * BLK + L, L)]
        zero = jnp.zeros((L,), jnp.float32)
        uu_v = uv0_v = uv1_v = vv0_v = vv1_v = zero
        for j in range(BLK):
            vv = vvecA if j < 8 else vvecB
            jj = j if j < 8 else j - 8
            qu = uvec[j] & 7
            q0 = vv[2 * jj] & 7
            q1 = vv[2 * jj + 1] & 7
            u = [ubuf[b, j, qu, pl.ds(c * L, L)] for c in range(EMB // L)]
            v0 = [vbuf[b, 2 * j, q0, pl.ds(c * L, L)] for c in range(EMB // L)]
            v1 = [vbuf[b, 2 * j + 1, q1, pl.ds(c * L, L)] for c in range(EMB // L)]
            suu = u[0] * u[0]
            suv0 = u[0] * v0[0]
            suv1 = u[0] * v1[0]
            svv0 = v0[0] * v0[0]
            svv1 = v1[0] * v1[0]
            for c in range(1, EMB // L):
                suu += u[c] * u[c]
                suv0 += u[c] * v0[c]
                suv1 += u[c] * v1[c]
                svv0 += v0[c] * v0[c]
                svv1 += v1[c] * v1[c]
            m = lane == j
            uu_v = jnp.where(m, allred(suu), uu_v)
            uv0_v = jnp.where(m, allred(suv0), uv0_v)
            uv1_v = jnp.where(m, allred(suv1), uv1_v)
            vv0_v = jnp.where(m, allred(svv0), vv0_v)
            vv1_v = jnp.where(m, allred(svv1), vv1_v)
        uu_a[pl.ds(t * BLK, L)] = uu_v
        uv0_a[pl.ds(t * BLK, L)] = uv0_v
        uv1_a[pl.ds(t * BLK, L)] = uv1_v
        vv0_a[pl.ds(t * BLK, L)] = vv0_v
        vv1_a[pl.ds(t * BLK, L)] = vv1_v
        return 0

    lax.fori_loop(0, NBLK, block, 0, unroll=False)

    def norm_loss(k, acc):
        uu = jnp.maximum(uu_a[pl.ds(k * L, L)], _EPS2)
        vv0 = jnp.maximum(vv0_a[pl.ds(k * L, L)], _EPS2)
        vv1 = jnp.maximum(vv1_a[pl.ds(k * L, L)], _EPS2)
        ru = _rsqrt_nr(uu)
        p0 = uv0_a[pl.ds(k * L, L)] * ru * _rsqrt_nr(vv0)
        p1 = uv1_a[pl.ds(k * L, L)] * ru * _rsqrt_nr(vv1)
        e0 = p0 - 1.0
        return acc + e0 * e0 + p1 * p1

    acc = lax.fori_loop(0, BPW // L, norm_loss,
                        jnp.zeros((L,), jnp.float32), unroll=False)
    acc_v[...] = acc
    pltpu.sync_copy(acc_v, out_hbm.at[wid])


@jax.jit
def kernel(uid, items, U, V):
    vid = items.reshape(-1)

    mf = pl.kernel(
        _mf_body,
        out_type=jax.ShapeDtypeStruct((NW, L), jnp.float32),
        mesh=plsc.VectorSubcoreMesh(core_axis_name="c", subcore_axis_name="s"),
        scratch_types=[
            pltpu.VMEM((BPW,), jnp.int32),
            pltpu.VMEM((2 * BPW,), jnp.int32),
            pltpu.VMEM((2, BLK, 8, EMB), jnp.float32),
            pltpu.VMEM((2, 2 * BLK, 8, EMB), jnp.float32),
            pltpu.VMEM((BPW,), jnp.float32),
            pltpu.VMEM((BPW,), jnp.float32),
            pltpu.VMEM((BPW,), jnp.float32),
            pltpu.VMEM((BPW,), jnp.float32),
            pltpu.VMEM((BPW,), jnp.float32),
            pltpu.VMEM((L,), jnp.float32),
            pltpu.SemaphoreType.DMA,
            pltpu.SemaphoreType.DMA,
        ],
        compiler_params=pltpu.CompilerParams(use_tc_tiling_on_sc=True),
    )
    partials = mf(uid, vid, U, V)
    return jnp.sum(partials) / jnp.float32(2 * BATCH)


# speedup vs baseline: 1.9625x; 1.4170x over previous
import jax
import jax.numpy as jnp
from jax import lax
from jax.experimental import pallas as pl
from jax.experimental.pallas import tpu as pltpu
from jax.experimental.pallas import tpu_sc as plsc

NC = 2
NS = 16
L = 16
NW = NC * NS

BATCH = 16384
EMB = 64
BPW = BATCH // NW
BLK = 16
NBLK = BPW // BLK

_EPS2 = 1e-24


def _rsqrt_nr(x):
    i = lax.bitcast_convert_type(x, jnp.int32)
    i = jnp.int32(0x5F3759DF) - (i >> 1)
    y = lax.bitcast_convert_type(i, jnp.float32)
    for _ in range(3):
        y = y * (1.5 - 0.5 * x * y * y)
    return y


def _mf_body(uid_hbm, vid_hbm, u_hbm, v_hbm, out_hbm,
             idxu, idxv, ubuf, vbuf, uu_a, uv0_a, uv1_a, vv0_a, vv1_a,
             acc_v, sem0, sem1):
    wid = lax.axis_index("s") * NC + lax.axis_index("c")

    pltpu.sync_copy(uid_hbm.at[pl.ds(wid * BPW, BPW)], idxu)
    pltpu.sync_copy(vid_hbm.at[pl.ds(wid * 2 * BPW, 2 * BPW)], idxv)

    lane = lax.iota(jnp.int32, L)
    perm = [lane ^ s for s in (8, 4, 2, 1)]

    def allred(x):
        for idx in perm:
            x = x + x[idx]
        return x

    def issue(t, b, sem):
        uvec = idxu[pl.ds(t * BLK, L)]
        vvecA = idxv[pl.ds(t * 2 * BLK, L)]
        vvecB = idxv[pl.ds(t * 2 * BLK + L, L)]
        for j in range(BLK):
            pltpu.async_copy(u_hbm.at[uvec[j] >> 3], ubuf.at[b, j], sem)
        for j in range(BLK):
            vv = vvecA if j < 8 else vvecB
            jj = j if j < 8 else j - 8
            for p in range(2):
                pltpu.async_copy(v_hbm.at[vv[2 * jj + p] >> 3],
                                 vbuf.at[b, 2 * j + p], sem)

    def drain(b, sem):
        pltpu.make_async_copy(u_hbm.at[pl.ds(0, BLK)], ubuf.at[b], sem).wait()
        pltpu.make_async_copy(v_hbm.at[pl.ds(0, 2 * BLK)], vbuf.at[b],
                              sem).wait()

    issue(0, 0, sem0)

    def block(t, _):
        @pl.when(t + 1 < NBLK)
        def _():
            @pl.when((t & 1) == 0)
            def _():
                issue(t + 1, 1, sem1)

            @pl.when((t & 1) == 1)
            def _():
                issue(t + 1, 0, sem0)

        @pl.when((t & 1) == 0)
        def _():
            drain(0, sem0)

        @pl.when((t & 1) == 1)
        def _():
            drain(1, sem1)

        b = t & 1
        uvec = idxu[pl.ds(t * BLK, L)]
        vvecA = idxv[pl.ds(t * 2 * BLK, L)]
        vvecB = idxv[pl.ds(t * 2 * BLK + L, L)]
        zero = jnp.zeros((L,), jnp.float32)
        uu_v = uv0_v = uv1_v = vv0_v = vv1_v = zero
        for j in range(BLK):
            vv = vvecA if j < 8 else vvecB
            jj = j if j < 8 else j - 8
            qu = uvec[j] & 7
            q0 = vv[2 * jj] & 7
            q1 = vv[2 * jj + 1] & 7
            u = [ubuf[b, j, qu, pl.ds(c * L, L)] for c in range(EMB // L)]
            v0 = [vbuf[b, 2 * j, q0, pl.ds(c * L, L)] for c in range(EMB // L)]
            v1 = [vbuf[b, 2 * j + 1, q1, pl.ds(c * L, L)] for c in range(EMB // L)]
            suu = u[0] * u[0]
            suv0 = u[0] * v0[0]
            suv1 = u[0] * v1[0]
            svv0 = v0[0] * v0[0]
            svv1 = v1[0] * v1[0]
            for c in range(1, EMB // L):
                suu += u[c] * u[c]
                suv0 += u[c] * v0[c]
                suv1 += u[c] * v1[c]
                svv0 += v0[c] * v0[c]
                svv1 += v1[c] * v1[c]
            m = lane == j
            uu_v = jnp.where(m, allred(suu), uu_v)
            uv0_v = jnp.where(m, allred(suv0), uv0_v)
            uv1_v = jnp.where(m, allred(suv1), uv1_v)
            vv0_v = jnp.where(m, allred(svv0), vv0_v)
            vv1_v = jnp.where(m, allred(svv1), vv1_v)
        uu_a[pl.ds(t * BLK, L)] = uu_v
        uv0_a[pl.ds(t * BLK, L)] = uv0_v
        uv1_a[pl.ds(t * BLK, L)] = uv1_v
        vv0_a[pl.ds(t * BLK, L)] = vv0_v
        vv1_a[pl.ds(t * BLK, L)] = vv1_v
        return 0

    lax.fori_loop(0, NBLK, block, 0, unroll=False)

    def norm_loss(k, acc):
        uu = jnp.maximum(uu_a[pl.ds(k * L, L)], _EPS2)
        vv0 = jnp.maximum(vv0_a[pl.ds(k * L, L)], _EPS2)
        vv1 = jnp.maximum(vv1_a[pl.ds(k * L, L)], _EPS2)
        ru = _rsqrt_nr(uu)
        p0 = uv0_a[pl.ds(k * L, L)] * ru * _rsqrt_nr(vv0)
        p1 = uv1_a[pl.ds(k * L, L)] * ru * _rsqrt_nr(vv1)
        e0 = p0 - 1.0
        return acc + e0 * e0 + p1 * p1

    acc = lax.fori_loop(0, BPW // L, norm_loss,
                        jnp.zeros((L,), jnp.float32), unroll=False)
    acc_v[...] = acc
    pltpu.sync_copy(acc_v, out_hbm.at[wid])


@jax.jit
def kernel(uid, items, U, V):
    vid = items.reshape(-1)
    u3 = U.reshape(100000 // 8, 8, EMB)
    v3 = V.reshape(1000000 // 8, 8, EMB)

    mf = pl.kernel(
        _mf_body,
        out_type=jax.ShapeDtypeStruct((NW, L), jnp.float32),
        mesh=plsc.VectorSubcoreMesh(core_axis_name="c", subcore_axis_name="s"),
        scratch_types=[
            pltpu.VMEM((BPW,), jnp.int32),
            pltpu.VMEM((2 * BPW,), jnp.int32),
            pltpu.VMEM((2, BLK, 8, EMB), jnp.float32),
            pltpu.VMEM((2, 2 * BLK, 8, EMB), jnp.float32),
            pltpu.VMEM((BPW,), jnp.float32),
            pltpu.VMEM((BPW,), jnp.float32),
            pltpu.VMEM((BPW,), jnp.float32),
            pltpu.VMEM((BPW,), jnp.float32),
            pltpu.VMEM((BPW,), jnp.float32),
            pltpu.VMEM((L,), jnp.float32),
            pltpu.SemaphoreType.DMA,
            pltpu.SemaphoreType.DMA,
        ],
        compiler_params=pltpu.CompilerParams(use_tc_tiling_on_sc=True),
    )
    partials = mf(uid, vid, u3, v3)
    return jnp.sum(partials) / jnp.float32(2 * BATCH)
